# baseline (speedup 1.0000x reference)
import jax
import jax.numpy as jnp
from jax.experimental import pallas as pl
from jax.experimental.pallas import tpu as pltpu

_L1 = 892
_B1 = _L1 - 1
_C1 = _B1 - 32
_L2 = 595
_B2 = _L2 - 2
_C2 = _B2 - 64
_NB = 128


def _lenet_kernel(x_ref, w1_ref, b1_ref, w2_ref, b2_ref,
                  f1w_ref, f1b_ref, f2w_ref, f2b_ref, f3w_ref, f3b_ref,
                  out_ref, p1_ref, p2_ref, c1_ref):
    B = _NB
    xf = jnp.transpose(x_ref[...].astype(jnp.bfloat16)).reshape(3, 1024 * B)
    for i in range(5):
        p1_ref[3 * i:3 * i + 3, :] = xf[:, 32 * i * B:(32 * i + 896) * B]
    y1 = jnp.dot(w1_ref[...], p1_ref[...],
                 preferred_element_type=jnp.float32)
    a1 = y1[0:6, 0:_L1 * B]
    for j in range(1, 5):
        a1 = a1 + y1[8 * j:8 * j + 6, j * B:(j + _L1) * B]
    a1 = jnp.maximum(a1 + b1_ref[...], 0.0)
    m1 = jnp.maximum(a1[:, 0:_B1 * B], a1[:, B:(_B1 + 1) * B])
    c1_ref[...] = jnp.maximum(m1[:, 0:_C1 * B],
                              m1[:, 32 * B:(32 + _C1) * B]).astype(jnp.bfloat16)

    for k in range(25):
        i, j = divmod(k, 5)
        off = 64 * i + 2 * j
        p2_ref[6 * k:6 * k + 6, :] = c1_ref[:, off * B:(off + _L2) * B]
    a2 = jnp.dot(w2_ref[...], p2_ref[...],
                 preferred_element_type=jnp.float32)
    a2 = jnp.maximum(a2 + b2_ref[...], 0.0)
    m2 = jnp.maximum(a2[:, 0:_B2 * B], a2[:, 2 * B:(_B2 + 2) * B])
    c2 = jnp.maximum(m2[:, 0:_C2 * B], m2[:, 64 * B:(64 + _C2) * B])

    feats = []
    for p in range(25):
        yq, xq = divmod(p, 5)
        sp = 128 * yq + 4 * xq
        feats.append(c2[:, sp * B:(sp + 1) * B])
    feat = jnp.concatenate(feats, axis=0)

    dn = (((0,), (0,)), ((), ()))
    h = jnp.maximum(jax.lax.dot_general(f1w_ref[...], feat, dn,
                                        preferred_element_type=jnp.float32)
                    + f1b_ref[...], 0.0)
    h = jnp.maximum(jax.lax.dot_general(f2w_ref[...], h, dn,
                                        preferred_element_type=jnp.float32)
                    + f2b_ref[...], 0.0)
    res = jax.lax.dot_general(f3w_ref[...], h, dn,
                              preferred_element_type=jnp.float32) + f3b_ref[...]
    out_ref[...] = jnp.transpose(res)


def kernel(conv1_w, conv1_b, conv2_w, conv2_b,
           fc1_w, fc1_b, fc2_w, fc2_b, fc3_w, fc3_b, x):
    n = x.shape[0]
    g = n // _NB
    xt = x.reshape(n, 3 * 1024)

    w1j = conv1_w.transpose(3, 0, 2, 1).reshape(5, 6, 15)
    w1 = jnp.zeros((5, 8, 15), jnp.float32).at[:, 0:6, :].set(w1j)
    w1 = w1.reshape(40, 15).astype(jnp.bfloat16)
    b1 = conv1_b.reshape(6, 1)
    w2 = conv2_w.transpose(0, 2, 3, 1).reshape(16, 150).astype(jnp.bfloat16)
    b2 = conv2_b.reshape(16, 1)
    f1w = fc1_w.reshape(16, 25, 100).transpose(1, 0, 2).reshape(400, 100)
    f1b = fc1_b.reshape(100, 1)
    f2w = fc2_w
    f2b = fc2_b.reshape(100, 1)
    f3w = fc3_w
    f3b = fc3_b.reshape(10, 1)

    def whole(a):
        nd = a.ndim
        return pl.BlockSpec(a.shape, lambda i, _nd=nd: (0,) * _nd)

    cls = getattr(pltpu, "CompilerParams", None) or getattr(pltpu, "TPUCompilerParams", None)
    cparams = cls(dimension_semantics=("parallel",)) if cls is not None else None

    out = pl.pallas_call(
        _lenet_kernel,
        out_shape=jax.ShapeDtypeStruct((n, 10), jnp.float32),
        grid=(g,),
        in_specs=[
            pl.BlockSpec((_NB, 3 * 1024), lambda i: (i, 0)),
            whole(w1), whole(b1), whole(w2), whole(b2),
            whole(f1w), whole(f1b), whole(f2w), whole(f2b),
            whole(f3w), whole(f3b),
        ],
        out_specs=pl.BlockSpec((_NB, 10), lambda i: (i, 0)),
        scratch_shapes=[
            pltpu.VMEM((15, 896 * _NB), jnp.bfloat16),
            pltpu.VMEM((150, _L2 * _NB), jnp.bfloat16),
            pltpu.VMEM((6, _C1 * _NB), jnp.bfloat16),
        ],
        compiler_params=cparams,
    )(xt, w1, b1, w2, b2, f1w, f1b, f2w, f2b, f3w, f3b)
    return out


# speedup vs baseline: 1.0289x; 1.0289x over previous
import jax
import jax.numpy as jnp
from jax.experimental import pallas as pl
from jax.experimental.pallas import tpu as pltpu

_L1 = 892
_B1 = _L1 - 1
_C1 = _B1 - 32
_L2 = 595
_B2 = _L2 - 2
_C2 = _B2 - 64
_NB = 128


def _lenet_kernel(x_ref, w1_ref, b1_ref, w2_ref, b2_ref,
                  f1w_ref, f1b_ref, f2w_ref, f2b_ref, f3w_ref, f3b_ref,
                  out_ref, p1_ref, p2_ref, c1_ref):
    B = _NB
    xf = jnp.transpose(x_ref[...]).reshape(3, 1024 * B)
    for i in range(5):
        p1_ref[3 * i:3 * i + 3, :] = xf[:, 32 * i * B:(32 * i + 896) * B]
    y1 = jnp.dot(w1_ref[...], p1_ref[...],
                 preferred_element_type=jnp.float32)
    a1 = y1[0:6, 0:_L1 * B]
    for j in range(1, 5):
        a1 = a1 + y1[8 * j:8 * j + 6, j * B:(j + _L1) * B]
    a1 = jnp.maximum(a1 + b1_ref[...], 0.0)
    m1 = jnp.maximum(a1[:, 0:_B1 * B], a1[:, B:(_B1 + 1) * B])
    c1_ref[...] = jnp.maximum(m1[:, 0:_C1 * B],
                              m1[:, 32 * B:(32 + _C1) * B]).astype(jnp.bfloat16)

    for k in range(25):
        i, j = divmod(k, 5)
        off = 64 * i + 2 * j
        p2_ref[6 * k:6 * k + 6, :] = c1_ref[:, off * B:(off + _L2) * B]
    a2 = jnp.dot(w2_ref[...], p2_ref[...],
                 preferred_element_type=jnp.float32)
    a2 = jnp.maximum(a2 + b2_ref[...], 0.0)
    m2 = jnp.maximum(a2[:, 0:_B2 * B], a2[:, 2 * B:(_B2 + 2) * B])
    c2 = jnp.maximum(m2[:, 0:_C2 * B], m2[:, 64 * B:(64 + _C2) * B])

    feats = []
    for p in range(25):
        yq, xq = divmod(p, 5)
        sp = 128 * yq + 4 * xq
        feats.append(c2[:, sp * B:(sp + 1) * B])
    feat = jnp.concatenate(feats, axis=0)

    h = jnp.maximum(jnp.dot(f1w_ref[...], feat,
                            preferred_element_type=jnp.float32) + f1b_ref[...], 0.0)
    h = jnp.maximum(jnp.dot(f2w_ref[...], h,
                            preferred_element_type=jnp.float32) + f2b_ref[...], 0.0)
    out_ref[...] = jnp.dot(f3w_ref[...], h,
                           preferred_element_type=jnp.float32) + f3b_ref[...]


def kernel(conv1_w, conv1_b, conv2_w, conv2_b,
           fc1_w, fc1_b, fc2_w, fc2_b, fc3_w, fc3_b, x):
    n = x.shape[0]
    g = n // _NB
    xt = x.reshape(n, 3 * 1024).astype(jnp.bfloat16)

    w1j = conv1_w.transpose(3, 0, 2, 1).reshape(5, 6, 15)
    w1 = jnp.zeros((5, 8, 15), jnp.float32).at[:, 0:6, :].set(w1j)
    w1 = w1.reshape(40, 15).astype(jnp.bfloat16)
    b1 = conv1_b.reshape(6, 1)
    w2 = conv2_w.transpose(0, 2, 3, 1).reshape(16, 150).astype(jnp.bfloat16)
    b2 = conv2_b.reshape(16, 1)
    f1w = fc1_w.reshape(16, 25, 100).transpose(2, 1, 0).reshape(100, 400)
    f1b = fc1_b.reshape(100, 1)
    f2w = fc2_w.T
    f2b = fc2_b.reshape(100, 1)
    f3w = fc3_w.T
    f3b = fc3_b.reshape(10, 1)

    def whole(a):
        nd = a.ndim
        return pl.BlockSpec(a.shape, lambda i, _nd=nd: (0,) * _nd)

    cls = getattr(pltpu, "CompilerParams", None) or getattr(pltpu, "TPUCompilerParams", None)
    cparams = cls(dimension_semantics=("parallel",)) if cls is not None else None

    out = pl.pallas_call(
        _lenet_kernel,
        out_shape=jax.ShapeDtypeStruct((10, n), jnp.float32),
        grid=(g,),
        in_specs=[
            pl.BlockSpec((_NB, 3 * 1024), lambda i: (i, 0)),
            whole(w1), whole(b1), whole(w2), whole(b2),
            whole(f1w), whole(f1b), whole(f2w), whole(f2b),
            whole(f3w), whole(f3b),
        ],
        out_specs=pl.BlockSpec((10, _NB), lambda i: (0, i)),
        scratch_shapes=[
            pltpu.VMEM((15, 896 * _NB), jnp.bfloat16),
            pltpu.VMEM((150, _L2 * _NB), jnp.bfloat16),
            pltpu.VMEM((6, _C1 * _NB), jnp.bfloat16),
        ],
        compiler_params=cparams,
    )(xt, w1, b1, w2, b2, f1w, f1b, f2w, f2b, f3w, f3b)
    return out.T


# speedup vs baseline: 1.2394x; 1.2047x over previous
import jax
import jax.numpy as jnp
from jax.experimental import pallas as pl
from jax.experimental.pallas import tpu as pltpu

_L1 = 892
_B1 = _L1 - 1
_C1 = _B1 - 32
_L2 = 595
_B2 = _L2 - 2
_C2 = _B2 - 64
_NB = 128


def _lenet_kernel(x_ref, w1_ref, b1_ref, w2_ref, b2_ref,
                  f1w_ref, f1b_ref, f2w_ref, f2b_ref, f3w_ref, f3b_ref,
                  out_ref, p1_ref, p2_ref, c1_ref):
    B = _NB
    xf = x_ref[...].reshape(3, 1024 * B).astype(jnp.bfloat16)
    for i in range(5):
        p1_ref[3 * i:3 * i + 3, :] = xf[:, 32 * i * B:(32 * i + 896) * B]
    y1 = jnp.dot(w1_ref[...], p1_ref[...],
                 preferred_element_type=jnp.float32)
    a1 = y1[0:6, 0:_L1 * B]
    for j in range(1, 5):
        a1 = a1 + y1[8 * j:8 * j + 6, j * B:(j + _L1) * B]
    a1 = jnp.maximum(a1 + b1_ref[...], 0.0)
    m1 = jnp.maximum(a1[:, 0:_B1 * B], a1[:, B:(_B1 + 1) * B])
    c1_ref[...] = jnp.maximum(m1[:, 0:_C1 * B],
                              m1[:, 32 * B:(32 + _C1) * B]).astype(jnp.bfloat16)

    for k in range(25):
        i, j = divmod(k, 5)
        off = 64 * i + 2 * j
        p2_ref[6 * k:6 * k + 6, :] = c1_ref[:, off * B:(off + _L2) * B]
    a2 = jnp.dot(w2_ref[...], p2_ref[...],
                 preferred_element_type=jnp.float32)
    a2 = jnp.maximum(a2 + b2_ref[...], 0.0)
    m2 = jnp.maximum(a2[:, 0:_B2 * B], a2[:, 2 * B:(_B2 + 2) * B])
    c2 = jnp.maximum(m2[:, 0:_C2 * B], m2[:, 64 * B:(64 + _C2) * B])

    feats = []
    for p in range(25):
        yq, xq = divmod(p, 5)
        sp = 128 * yq + 4 * xq
        feats.append(c2[:, sp * B:(sp + 1) * B])
    feat = jnp.concatenate(feats, axis=0)

    h = jnp.maximum(jnp.dot(f1w_ref[...], feat,
                            preferred_element_type=jnp.float32) + f1b_ref[...], 0.0)
    h = jnp.maximum(jnp.dot(f2w_ref[...], h,
                            preferred_element_type=jnp.float32) + f2b_ref[...], 0.0)
    out_ref[...] = jnp.dot(f3w_ref[...], h,
                           preferred_element_type=jnp.float32) + f3b_ref[...]


def kernel(conv1_w, conv1_b, conv2_w, conv2_b,
           fc1_w, fc1_b, fc2_w, fc2_b, fc3_w, fc3_b, x):
    n = x.shape[0]
    g = n // _NB
    xt = x.reshape(n, 3 * 1024).T

    w1j = conv1_w.transpose(3, 0, 2, 1).reshape(5, 6, 15)
    w1 = jnp.zeros((5, 8, 15), jnp.float32).at[:, 0:6, :].set(w1j)
    w1 = w1.reshape(40, 15).astype(jnp.bfloat16)
    b1 = conv1_b.reshape(6, 1)
    w2 = conv2_w.transpose(0, 2, 3, 1).reshape(16, 150).astype(jnp.bfloat16)
    b2 = conv2_b.reshape(16, 1)
    f1w = fc1_w.reshape(16, 25, 100).transpose(2, 1, 0).reshape(100, 400)
    f1b = fc1_b.reshape(100, 1)
    f2w = fc2_w.T
    f2b = fc2_b.reshape(100, 1)
    f3w = fc3_w.T
    f3b = fc3_b.reshape(10, 1)

    def whole(a):
        nd = a.ndim
        return pl.BlockSpec(a.shape, lambda i, _nd=nd: (0,) * _nd)

    cls = getattr(pltpu, "CompilerParams", None) or getattr(pltpu, "TPUCompilerParams", None)
    cparams = cls(dimension_semantics=("parallel",)) if cls is not None else None

    out = pl.pallas_call(
        _lenet_kernel,
        out_shape=jax.ShapeDtypeStruct((10, n), jnp.float32),
        grid=(g,),
        in_specs=[
            pl.BlockSpec((3 * 1024, _NB), lambda i: (0, i)),
            whole(w1), whole(b1), whole(w2), whole(b2),
            whole(f1w), whole(f1b), whole(f2w), whole(f2b),
            whole(f3w), whole(f3b),
        ],
        out_specs=pl.BlockSpec((10, _NB), lambda i: (0, i)),
        scratch_shapes=[
            pltpu.VMEM((15, 896 * _NB), jnp.bfloat16),
            pltpu.VMEM((150, _L2 * _NB), jnp.bfloat16),
            pltpu.VMEM((6, _C1 * _NB), jnp.bfloat16),
        ],
        compiler_params=cparams,
    )(xt, w1, b1, w2, b2, f1w, f1b, f2w, f2b, f3w, f3b)
    return out.T


# speedup vs baseline: 1.3052x; 1.0531x over previous
import jax
import jax.numpy as jnp
from jax.experimental import pallas as pl
from jax.experimental.pallas import tpu as pltpu

_L1 = 892
_B1 = _L1 - 1
_C1 = _B1 - 32
_L2 = 595
_B2 = _L2 - 2
_C2 = _B2 - 64
_NB = 128


def _lenet_kernel(x_ref, w1_ref, w2_ref, f1w_ref, f2w_ref, f3w_ref, bias_ref,
                  out_ref, p1_ref, p2_ref, c1_ref):
    B = _NB
    xf = x_ref[...].reshape(3, 1024 * B).astype(jnp.bfloat16)
    for i in range(5):
        p1_ref[3 * i:3 * i + 3, :] = xf[:, 32 * i * B:(32 * i + 896) * B]
    y1 = jnp.dot(w1_ref[...], p1_ref[...],
                 preferred_element_type=jnp.float32)
    a1 = y1[0:6, 0:_L1 * B]
    for j in range(1, 5):
        a1 = a1 + y1[8 * j:8 * j + 6, j * B:(j + _L1) * B]
    a1 = jnp.maximum(a1 + bias_ref[0:6], 0.0)
    m1 = jnp.maximum(a1[:, 0:_B1 * B], a1[:, B:(_B1 + 1) * B])
    c1_ref[...] = jnp.maximum(m1[:, 0:_C1 * B],
                              m1[:, 32 * B:(32 + _C1) * B]).astype(jnp.bfloat16)

    for k in range(25):
        i, j = divmod(k, 5)
        off = 64 * i + 2 * j
        p2_ref[6 * k:6 * k + 6, :] = c1_ref[:, off * B:(off + _L2) * B]
    a2 = jnp.dot(w2_ref[...], p2_ref[...],
                 preferred_element_type=jnp.float32)
    a2 = jnp.maximum(a2 + bias_ref[6:22], 0.0)
    m2 = jnp.maximum(a2[:, 0:_B2 * B], a2[:, 2 * B:(_B2 + 2) * B])
    c2 = jnp.maximum(m2[:, 0:_C2 * B], m2[:, 64 * B:(64 + _C2) * B])

    feats = []
    for p in range(25):
        yq, xq = divmod(p, 5)
        sp = 128 * yq + 4 * xq
        feats.append(c2[:, sp * B:(sp + 1) * B])
    feat = jnp.concatenate(feats, axis=0)

    h = jnp.maximum(jnp.dot(f1w_ref[...], feat,
                            preferred_element_type=jnp.float32) + bias_ref[22:122], 0.0)
    h = jnp.maximum(jnp.dot(f2w_ref[...], h,
                            preferred_element_type=jnp.float32) + bias_ref[122:222], 0.0)
    out_ref[...] = jnp.dot(f3w_ref[...], h,
                           preferred_element_type=jnp.float32) + bias_ref[222:232]


def kernel(conv1_w, conv1_b, conv2_w, conv2_b,
           fc1_w, fc1_b, fc2_w, fc2_b, fc3_w, fc3_b, x):
    n = x.shape[0]
    g = n // _NB
    xt = x.reshape(n, 3 * 1024).T

    w1j = conv1_w.transpose(3, 0, 2, 1).reshape(5, 6, 15)
    w1 = jnp.zeros((5, 8, 15), jnp.float32).at[:, 0:6, :].set(w1j)
    w1 = w1.reshape(40, 15).astype(jnp.bfloat16)
    w2 = conv2_w.transpose(0, 2, 3, 1).reshape(16, 150).astype(jnp.bfloat16)
    f1w = fc1_w.reshape(16, 25, 100).transpose(2, 1, 0).reshape(100, 400)
    f2w = fc2_w.T
    f3w = fc3_w.T
    bias = jnp.concatenate([conv1_b, conv2_b, fc1_b, fc2_b,
                            fc3_b]).reshape(232, 1)

    def whole(a):
        nd = a.ndim
        return pl.BlockSpec(a.shape, lambda i, _nd=nd: (0,) * _nd)

    cls = getattr(pltpu, "CompilerParams", None) or getattr(pltpu, "TPUCompilerParams", None)
    cparams = cls(dimension_semantics=("parallel",)) if cls is not None else None

    out = pl.pallas_call(
        _lenet_kernel,
        out_shape=jax.ShapeDtypeStruct((10, n), jnp.float32),
        grid=(g,),
        in_specs=[
            pl.BlockSpec((3 * 1024, _NB), lambda i: (0, i)),
            whole(w1), whole(w2),
            whole(f1w), whole(f2w), whole(f3w), whole(bias),
        ],
        out_specs=pl.BlockSpec((10, _NB), lambda i: (0, i)),
        scratch_shapes=[
            pltpu.VMEM((15, 896 * _NB), jnp.bfloat16),
            pltpu.VMEM((150, _L2 * _NB), jnp.bfloat16),
            pltpu.VMEM((6, _C1 * _NB), jnp.bfloat16),
        ],
        compiler_params=cparams,
    )(xt, w1, w2, f1w, f2w, f3w, bias)
    return out.T
